# baseline (speedup 1.0000x reference)
import functools
import math

import jax
import jax.numpy as jnp
from jax.experimental import pallas as pl
from jax.experimental.pallas import tpu as pltpu

CHUNK = 2048


def _router_body(q_ref, k_ref, bias_ref, wq_ref, wk_ref, bs_ref,
                 out_ref, pq_ref, pk_ref, *, nb, nc, bs, d_head):
    c = pl.program_id(2)
    rows = CHUNK // bs

    qc = q_ref[0, 0].reshape(rows, bs, d_head)
    kc = k_ref[0, 0].reshape(rows, bs, d_head)
    inv = 1.0 / bs
    pq_ref[pl.ds(c * rows, rows), :] = qc.sum(axis=1) * inv
    pk_ref[pl.ds(c * rows, rows), :] = kc.sum(axis=1) * inv

    @pl.when(c == nc - 1)
    def _finish():
        qp = jnp.dot(pq_ref[...], wq_ref[...].T,
                     preferred_element_type=jnp.float32)
        kp = jnp.dot(pk_ref[...], wk_ref[...].T,
                     preferred_element_type=jnp.float32)
        scores = jnp.dot(qp, kp.T, preferred_element_type=jnp.float32)
        scores = scores / math.sqrt(d_head)
        scores = scores + bs_ref[0, 0] * bias_ref[0, 0]
        scaled = scores / 0.1

        target = 0.15 * nb
        lo = jnp.full((nb, 1), -10000.0, dtype=jnp.float32)
        hi = jnp.full((nb, 1), 10000.0, dtype=jnp.float32)
        for _ in range(20):
            mid = (lo + hi) * 0.5
            total = jnp.sum(jax.nn.sigmoid(scaled + mid), axis=1,
                            keepdims=True)
            below = total < target
            lo = jnp.where(below, mid, lo)
            hi = jnp.where(below, hi, mid)
        lam = (lo + hi) * 0.5
        out_ref[0, 0] = jax.nn.sigmoid(scaled + lam)


def kernel(q, k, pooled_bias, Wq, Wk, bias_scale, block_size):
    B, H, L, d_head = q.shape
    nb = pooled_bias.shape[2]
    bs = L // nb
    nc = L // CHUNK

    bias_scale = jnp.asarray(bias_scale, jnp.float32).reshape(1, 1)

    body = functools.partial(_router_body, nb=nb, nc=nc, bs=bs,
                             d_head=d_head)
    grid = (B, H, nc)
    out = pl.pallas_call(
        body,
        grid=grid,
        in_specs=[
            pl.BlockSpec((1, 1, CHUNK, d_head), lambda b, h, c: (b, h, c, 0)),
            pl.BlockSpec((1, 1, CHUNK, d_head), lambda b, h, c: (b, h, c, 0)),
            pl.BlockSpec((1, 1, nb, nb), lambda b, h, c: (0, h, 0, 0)),
            pl.BlockSpec((d_head, d_head), lambda b, h, c: (0, 0)),
            pl.BlockSpec((d_head, d_head), lambda b, h, c: (0, 0)),
            pl.BlockSpec((1, 1), lambda b, h, c: (0, 0)),
        ],
        out_specs=pl.BlockSpec((1, 1, nb, nb), lambda b, h, c: (b, h, 0, 0)),
        out_shape=jax.ShapeDtypeStruct((B, H, nb, nb), jnp.float32),
        scratch_shapes=[
            pltpu.VMEM((nb, d_head), jnp.float32),
            pltpu.VMEM((nb, d_head), jnp.float32),
        ],
    )(q, k, pooled_bias, Wq, Wk, bias_scale)
    return out


# speedup vs baseline: 1.1260x; 1.1260x over previous
import functools
import math

import jax
import jax.numpy as jnp
from jax.experimental import pallas as pl
from jax.experimental.pallas import tpu as pltpu

CHUNK = 2048


def _router_body(q_ref, k_ref, bias_ref, wq_ref, wk_ref, bs_ref,
                 out_ref, pq_ref, pk_ref, *, nb, nc, bs, d_head):
    c = pl.program_id(2)
    rows = CHUNK // bs

    qc = q_ref[0, 0].reshape(rows, bs, d_head)
    kc = k_ref[0, 0].reshape(rows, bs, d_head)
    inv = 1.0 / bs
    pq_ref[pl.ds(c * rows, rows), :] = qc.sum(axis=1) * inv
    pk_ref[pl.ds(c * rows, rows), :] = kc.sum(axis=1) * inv

    @pl.when(c == nc - 1)
    def _finish():
        qp = jnp.dot(pq_ref[...], wq_ref[...].T,
                     preferred_element_type=jnp.float32)
        kp = jnp.dot(pk_ref[...], wk_ref[...].T,
                     preferred_element_type=jnp.float32)
        scores = jnp.dot(qp, kp.T, preferred_element_type=jnp.float32)
        scores = scores / math.sqrt(d_head)
        scores = scores + bs_ref[0, 0] * bias_ref[0, 0]
        scaled = scores / 0.1

        target = 0.15 * nb
        c = math.log(0.15 / 0.85)
        a = c - jnp.max(scaled, axis=1, keepdims=True)
        b = c - jnp.min(scaled, axis=1, keepdims=True)
        x = (a + b) * 0.5
        for _ in range(6):
            sig = jax.nn.sigmoid(scaled + x)
            fx = jnp.sum(sig, axis=1, keepdims=True) - target
            dfx = jnp.sum(sig - sig * sig, axis=1, keepdims=True)
            neg = fx < 0.0
            a = jnp.where(neg, x, a)
            b = jnp.where(neg, b, x)
            xn = x - fx / jnp.maximum(dfx, 1e-12)
            bad = jnp.logical_or(xn < a, xn > b)
            x = jnp.where(bad, (a + b) * 0.5, xn)
        lo = jnp.full((nb, 1), -10000.0, dtype=jnp.float32)
        hi = jnp.full((nb, 1), 10000.0, dtype=jnp.float32)
        for _ in range(20):
            mid = (lo + hi) * 0.5
            below = mid < x
            lo = jnp.where(below, mid, lo)
            hi = jnp.where(below, hi, mid)
        lam = (lo + hi) * 0.5
        out_ref[0, 0] = jax.nn.sigmoid(scaled + lam)


def kernel(q, k, pooled_bias, Wq, Wk, bias_scale, block_size):
    B, H, L, d_head = q.shape
    nb = pooled_bias.shape[2]
    bs = L // nb
    nc = L // CHUNK

    bias_scale = jnp.asarray(bias_scale, jnp.float32).reshape(1, 1)

    body = functools.partial(_router_body, nb=nb, nc=nc, bs=bs,
                             d_head=d_head)
    grid = (B, H, nc)
    out = pl.pallas_call(
        body,
        grid=grid,
        in_specs=[
            pl.BlockSpec((1, 1, CHUNK, d_head), lambda b, h, c: (b, h, c, 0)),
            pl.BlockSpec((1, 1, CHUNK, d_head), lambda b, h, c: (b, h, c, 0)),
            pl.BlockSpec((1, 1, nb, nb), lambda b, h, c: (0, h, 0, 0)),
            pl.BlockSpec((d_head, d_head), lambda b, h, c: (0, 0)),
            pl.BlockSpec((d_head, d_head), lambda b, h, c: (0, 0)),
            pl.BlockSpec((1, 1), lambda b, h, c: (0, 0)),
        ],
        out_specs=pl.BlockSpec((1, 1, nb, nb), lambda b, h, c: (b, h, 0, 0)),
        out_shape=jax.ShapeDtypeStruct((B, H, nb, nb), jnp.float32),
        scratch_shapes=[
            pltpu.VMEM((nb, d_head), jnp.float32),
            pltpu.VMEM((nb, d_head), jnp.float32),
        ],
    )(q, k, pooled_bias, Wq, Wk, bias_scale)
    return out


# speedup vs baseline: 2.0121x; 1.7870x over previous
import functools
import math

import jax
import jax.numpy as jnp
from jax.experimental import pallas as pl
from jax.experimental.pallas import tpu as pltpu


def _router_body(q_ref, k_ref, bias_ref, wq_ref, wk_ref, bs_ref,
                 out_ref, *, nb, bs, d_head):
    inv = 1.0 / bs
    pq = q_ref[0, 0].reshape(nb, bs, d_head).sum(axis=1) * inv
    pk = k_ref[0, 0].reshape(nb, bs, d_head).sum(axis=1) * inv

    qp = jnp.dot(pq, wq_ref[...].T, preferred_element_type=jnp.float32)
    kp = jnp.dot(pk, wk_ref[...].T, preferred_element_type=jnp.float32)
    scores = jnp.dot(qp, kp.T, preferred_element_type=jnp.float32)
    scores = scores / math.sqrt(d_head)
    scores = scores + bs_ref[0, 0] * bias_ref[0, 0]
    scaled = scores / 0.1

    target = 0.15 * nb
    c = math.log(0.15 / 0.85)
    a = c - jnp.max(scaled, axis=1, keepdims=True)
    b = c - jnp.min(scaled, axis=1, keepdims=True)
    x = (a + b) * 0.5
    for _ in range(6):
        sig = jax.nn.sigmoid(scaled + x)
        fx = jnp.sum(sig, axis=1, keepdims=True) - target
        dfx = jnp.sum(sig - sig * sig, axis=1, keepdims=True)
        neg = fx < 0.0
        a = jnp.where(neg, x, a)
        b = jnp.where(neg, b, x)
        xn = x - fx / jnp.maximum(dfx, 1e-12)
        bad = jnp.logical_or(xn < a, xn > b)
        x = jnp.where(bad, (a + b) * 0.5, xn)
    lo = jnp.full((nb, 1), -10000.0, dtype=jnp.float32)
    hi = jnp.full((nb, 1), 10000.0, dtype=jnp.float32)
    for _ in range(20):
        mid = (lo + hi) * 0.5
        below = mid < x
        lo = jnp.where(below, mid, lo)
        hi = jnp.where(below, hi, mid)
    lam = (lo + hi) * 0.5
    out_ref[0, 0] = jax.nn.sigmoid(scaled + lam)


def kernel(q, k, pooled_bias, Wq, Wk, bias_scale, block_size):
    B, H, L, d_head = q.shape
    nb = pooled_bias.shape[2]
    bs = L // nb

    bias_scale = jnp.asarray(bias_scale, jnp.float32).reshape(1, 1)

    body = functools.partial(_router_body, nb=nb, bs=bs, d_head=d_head)
    out = pl.pallas_call(
        body,
        grid=(B, H),
        in_specs=[
            pl.BlockSpec((1, 1, L, d_head), lambda bi, h: (bi, h, 0, 0)),
            pl.BlockSpec((1, 1, L, d_head), lambda bi, h: (bi, h, 0, 0)),
            pl.BlockSpec((1, 1, nb, nb), lambda bi, h: (0, h, 0, 0)),
            pl.BlockSpec((d_head, d_head), lambda bi, h: (0, 0)),
            pl.BlockSpec((d_head, d_head), lambda bi, h: (0, 0)),
            pl.BlockSpec((1, 1), lambda bi, h: (0, 0)),
        ],
        out_specs=pl.BlockSpec((1, 1, nb, nb), lambda bi, h: (bi, h, 0, 0)),
        out_shape=jax.ShapeDtypeStruct((B, H, nb, nb), jnp.float32),
    )(q, k, pooled_bias, Wq, Wk, bias_scale)
    return out
